# baseline (speedup 1.0000x reference)
import functools

import jax
import jax.numpy as jnp
from jax import lax
from jax.experimental import pallas as pl
from jax.experimental.pallas import tpu as pltpu
from jax.experimental.pallas import tpu_sc as plsc

_LN_EPS = 1e-5
_NC = 2
_NS = 16
_NW = _NC * _NS
_IDXW = 128


def _repack_body(tt_ref, eye_ref, out_ref):
    x = tt_ref[...]
    xt = lax.dot_general(x, eye_ref[...], (((0,), (0,)), ((), ())),
                         preferred_element_type=jnp.float32)
    out_ref[...] = jnp.concatenate([xt, xt], axis=1)


def _repack(table_t, blk):
    d, v = table_t.shape
    return pl.pallas_call(
        _repack_body,
        grid=(pl.cdiv(v, blk),),
        in_specs=[
            pl.BlockSpec((d, blk), lambda i: (0, i)),
            pl.BlockSpec((d, d), lambda i: (0, 0)),
        ],
        out_specs=pl.BlockSpec((blk, 2 * d), lambda i: (i, 0)),
        out_shape=jax.ShapeDtypeStruct((v, 2 * d), jnp.float32),
    )(table_t, jnp.eye(d, dtype=jnp.float32))


def _gather_body(nchunks, ids_hbm, table_hbm, out_hbm, idx_v, rows_v, sem):
    wid = lax.axis_index("s") * _NC + lax.axis_index("c")

    def step(i, carry):
        ci = wid * nchunks + i
        pltpu.sync_copy(ids_hbm.at[ci], idx_v)
        for half in range(2):
            copies = []
            for j in range(4):
                cp = pltpu.make_async_copy(
                    table_hbm.at[idx_v.at[half * 4 + j]],
                    rows_v.at[pl.ds(j * _IDXW, _IDXW)],
                    sem,
                )
                cp.start()
                copies.append(cp)
            for cp in copies:
                cp.wait()
            off = pl.multiple_of((ci * 2 + half) * 512, 512)
            pltpu.sync_copy(rows_v, out_hbm.at[pl.ds(off, 512)])
        return carry

    lax.fori_loop(0, nchunks, step, 0)


def _sc_gather(ids3d, table128):
    n = ids3d.shape[0] * 1024
    nchunks = ids3d.shape[0] // _NW
    mesh = plsc.VectorSubcoreMesh(core_axis_name="c", subcore_axis_name="s")
    f = pl.kernel(
        functools.partial(_gather_body, nchunks),
        out_type=jax.ShapeDtypeStruct((n, 128), jnp.float32),
        mesh=mesh,
        scratch_types=[
            pltpu.VMEM((8, _IDXW), jnp.int32),
            pltpu.VMEM((512, 128), jnp.float32),
            pltpu.SemaphoreType.DMA,
        ],
    )
    return f(ids3d, table128)


def _lin_ln_t_body(w_ref, b_ref, g_ref, be_ref, emb_ref, out_ref):
    x = emb_ref[...]
    w = w_ref[...]
    y = lax.dot_general(w, x, (((1,), (1,)), ((), ())),
                        preferred_element_type=jnp.float32)
    y = y + b_ref[...]
    m = jnp.mean(y, axis=0, keepdims=True)
    c = y - m
    v = jnp.mean(c * c, axis=0, keepdims=True)
    r = (c * lax.rsqrt(v + _LN_EPS)) * g_ref[...] + be_ref[...]
    out_ref[...] = r[None]


def _lin_ln_t(emb, w128, b, gamma, beta, fields, bsz, blk):
    d = w128.shape[0]
    nb = bsz // blk
    return pl.pallas_call(
        _lin_ln_t_body,
        grid=(fields, nb),
        in_specs=[
            pl.BlockSpec((d, 2 * d), lambda f, i: (0, 0)),
            pl.BlockSpec((d, 1), lambda f, i: (0, 0)),
            pl.BlockSpec((d, 1), lambda f, i: (0, 0)),
            pl.BlockSpec((d, 1), lambda f, i: (0, 0)),
            pl.BlockSpec((blk, 2 * d), lambda f, i: (f * nb + i, 0)),
        ],
        out_specs=pl.BlockSpec((1, d, blk), lambda f, i: (f, 0, i)),
        out_shape=jax.ShapeDtypeStruct((fields, d, bsz), jnp.float32),
    )(w128, b.reshape(d, 1), gamma.reshape(d, 1), beta.reshape(d, 1), emb)


def kernel(concept_ids, table, W, b, gamma, beta):
    bsz, fields = concept_ids.shape
    d = table.shape[1]
    n = bsz * fields

    table128 = _repack(table.T, blk=512)

    ids = concept_ids.T.reshape(n).astype(jnp.int32)
    ids3d = ids.reshape(n // 1024, 8, _IDXW)

    emb = _sc_gather(ids3d, table128)

    w128 = jnp.concatenate([W, jnp.zeros_like(W)], axis=1)
    out_t = _lin_ln_t(emb, w128, b, gamma, beta, fields, bsz, blk=2048)
    return out_t.transpose(2, 0, 1)


# speedup vs baseline: 1.9649x; 1.9649x over previous
import functools

import jax
import jax.numpy as jnp
from jax import lax
from jax.experimental import pallas as pl
from jax.experimental.pallas import tpu as pltpu
from jax.experimental.pallas import tpu_sc as plsc

_LN_EPS = 1e-5
_NC = 2
_NS = 16
_NW = _NC * _NS
_IDXW = 128


def _repack_body(tt_ref, eye2_ref, out_ref):
    x = tt_ref[...]
    out_ref[...] = lax.dot_general(x, eye2_ref[...], (((0,), (0,)), ((), ())),
                                   preferred_element_type=jnp.float32)


def _repack(table_t, blk):
    d, v = table_t.shape
    eye2 = jnp.concatenate([jnp.eye(d, dtype=jnp.float32)] * 2, axis=1)
    return pl.pallas_call(
        _repack_body,
        grid=(pl.cdiv(v, blk),),
        in_specs=[
            pl.BlockSpec((d, blk), lambda i: (0, i)),
            pl.BlockSpec((d, 2 * d), lambda i: (0, 0)),
        ],
        out_specs=pl.BlockSpec((blk, 2 * d), lambda i: (i, 0)),
        out_shape=jax.ShapeDtypeStruct((v, 2 * d), jnp.float32),
    )(table_t, eye2)


def _gather_body(nchunks, ids_hbm, table_hbm, out_hbm, idx_v, rows_v, sem):
    wid = lax.axis_index("s") * _NC + lax.axis_index("c")

    def step(i, carry):
        ci = wid * nchunks + i
        pltpu.sync_copy(ids_hbm.at[ci], idx_v)
        for half in range(2):
            copies = []
            for j in range(4):
                cp = pltpu.make_async_copy(
                    table_hbm.at[idx_v.at[half * 4 + j]],
                    rows_v.at[pl.ds(j * _IDXW, _IDXW)],
                    sem,
                )
                cp.start()
                copies.append(cp)
            for cp in copies:
                cp.wait()
            off = pl.multiple_of((ci * 2 + half) * 512, 512)
            pltpu.sync_copy(rows_v, out_hbm.at[pl.ds(off, 512)])
        return carry

    lax.fori_loop(0, nchunks, step, 0)


def _sc_gather(ids3d, table128):
    n = ids3d.shape[0] * 1024
    nchunks = ids3d.shape[0] // _NW
    mesh = plsc.VectorSubcoreMesh(core_axis_name="c", subcore_axis_name="s")
    f = pl.kernel(
        functools.partial(_gather_body, nchunks),
        out_type=jax.ShapeDtypeStruct((n, 128), jnp.float32),
        mesh=mesh,
        scratch_types=[
            pltpu.VMEM((8, _IDXW), jnp.int32),
            pltpu.VMEM((512, 128), jnp.float32),
            pltpu.SemaphoreType.DMA,
        ],
    )
    return f(ids3d, table128)


def _lin_ln_t_body(w_ref, b_ref, g_ref, be_ref, emb_ref, out_ref):
    x = emb_ref[...]
    w = w_ref[...]
    y = lax.dot_general(w, x, (((1,), (1,)), ((), ())),
                        preferred_element_type=jnp.float32)
    y = y + b_ref[...]
    m = jnp.mean(y, axis=0, keepdims=True)
    c = y - m
    v = jnp.mean(c * c, axis=0, keepdims=True)
    r = (c * lax.rsqrt(v + _LN_EPS)) * g_ref[...] + be_ref[...]
    out_ref[...] = r[None]


def _lin_ln_t(emb, w128, b, gamma, beta, fields, bsz, blk):
    d = w128.shape[0]
    nb = bsz // blk
    return pl.pallas_call(
        _lin_ln_t_body,
        grid=(fields, nb),
        in_specs=[
            pl.BlockSpec((d, 2 * d), lambda f, i: (0, 0)),
            pl.BlockSpec((d, 1), lambda f, i: (0, 0)),
            pl.BlockSpec((d, 1), lambda f, i: (0, 0)),
            pl.BlockSpec((d, 1), lambda f, i: (0, 0)),
            pl.BlockSpec((blk, 2 * d), lambda f, i: (f * nb + i, 0)),
        ],
        out_specs=pl.BlockSpec((1, d, blk), lambda f, i: (f, 0, i)),
        out_shape=jax.ShapeDtypeStruct((fields, d, bsz), jnp.float32),
    )(w128, b.reshape(d, 1), gamma.reshape(d, 1), beta.reshape(d, 1), emb)


def kernel(concept_ids, table, W, b, gamma, beta):
    bsz, fields = concept_ids.shape
    d = table.shape[1]
    n = bsz * fields

    table128 = _repack(table.T, blk=2048)

    ids = concept_ids.T.reshape(n).astype(jnp.int32)
    ids3d = ids.reshape(n // 1024, 8, _IDXW)

    emb = _sc_gather(ids3d, table128)

    w128 = jnp.concatenate([W, jnp.zeros_like(W)], axis=1)
    out_t = _lin_ln_t(emb, w128, b, gamma, beta, fields, bsz, blk=2048)
    return out_t.transpose(2, 0, 1)


# speedup vs baseline: 2.5214x; 1.2832x over previous
import functools

import jax
import jax.numpy as jnp
from jax import lax
from jax.experimental import pallas as pl
from jax.experimental.pallas import tpu as pltpu
from jax.experimental.pallas import tpu_sc as plsc

_LN_EPS = 1e-5
_NC = 2
_NS = 16
_NW = _NC * _NS
_IDXW = 128


def _repack_body(tt_ref, eye2_ref, out_ref):
    x = tt_ref[...]
    out_ref[...] = lax.dot_general(x, eye2_ref[...], (((0,), (0,)), ((), ())),
                                   preferred_element_type=jnp.float32)


def _repack(table_t, blk):
    d, v = table_t.shape
    eye2 = jnp.concatenate([jnp.eye(d, dtype=jnp.float32)] * 2, axis=1)
    return pl.pallas_call(
        _repack_body,
        grid=(pl.cdiv(v, blk),),
        in_specs=[
            pl.BlockSpec((d, blk), lambda i: (0, i)),
            pl.BlockSpec((d, 2 * d), lambda i: (0, 0)),
        ],
        out_specs=pl.BlockSpec((blk, 2 * d), lambda i: (i, 0)),
        out_shape=jax.ShapeDtypeStruct((v, 2 * d), jnp.float32),
    )(table_t, eye2)


def _gather_body(nchunks, ids_hbm, table_hbm, out_hbm, idx_v, rows_v, sem):
    wid = lax.axis_index("s") * _NC + lax.axis_index("c")

    def step(i, carry):
        ci = wid * nchunks + i
        pltpu.sync_copy(ids_hbm.at[ci], idx_v)
        for half in range(2):
            copies = []
            for j in range(4):
                cp = pltpu.make_async_copy(
                    table_hbm.at[idx_v.at[half * 4 + j]],
                    rows_v.at[pl.ds(j * _IDXW, _IDXW)],
                    sem,
                )
                cp.start()
                copies.append(cp)
            for cp in copies:
                cp.wait()
            off = pl.multiple_of((ci * 2 + half) * 512, 512)
            pltpu.sync_copy(rows_v, out_hbm.at[pl.ds(off, 512)])
        return carry

    lax.fori_loop(0, nchunks, step, 0)


def _sc_gather(ids3d, table128):
    n = ids3d.shape[0] * 1024
    nchunks = ids3d.shape[0] // _NW
    mesh = plsc.VectorSubcoreMesh(core_axis_name="c", subcore_axis_name="s")
    f = pl.kernel(
        functools.partial(_gather_body, nchunks),
        out_type=jax.ShapeDtypeStruct((n, 128), jnp.float32),
        mesh=mesh,
        scratch_types=[
            pltpu.VMEM((8, _IDXW), jnp.int32),
            pltpu.VMEM((512, 128), jnp.float32),
            pltpu.SemaphoreType.DMA,
        ],
    )
    return f(ids3d, table128)


def _lin_ln_t_body(w_ref, b_ref, g_ref, be_ref, emb_ref, out_ref):
    x = emb_ref[...]
    w = w_ref[...]
    y = lax.dot_general(w, x, (((1,), (1,)), ((), ())),
                        preferred_element_type=jnp.float32)
    y = y + b_ref[...]
    m = jnp.mean(y, axis=0, keepdims=True)
    c = y - m
    v = jnp.mean(c * c, axis=0, keepdims=True)
    r = (c * lax.rsqrt(v + _LN_EPS)) * g_ref[...] + be_ref[...]
    out_ref[...] = r[None]


def _lin_ln_t(emb, w128, b, gamma, beta, fields, bsz, blk):
    d = w128.shape[0]
    nb = bsz // blk
    return pl.pallas_call(
        _lin_ln_t_body,
        grid=(fields, nb),
        in_specs=[
            pl.BlockSpec((d, 2 * d), lambda f, i: (0, 0)),
            pl.BlockSpec((d, 1), lambda f, i: (0, 0)),
            pl.BlockSpec((d, 1), lambda f, i: (0, 0)),
            pl.BlockSpec((d, 1), lambda f, i: (0, 0)),
            pl.BlockSpec((blk, 2 * d), lambda f, i: (f * nb + i, 0)),
        ],
        out_specs=pl.BlockSpec((1, d, blk), lambda f, i: (f, 0, i)),
        out_shape=jax.ShapeDtypeStruct((fields, d, bsz), jnp.float32),
    )(w128, b.reshape(d, 1), gamma.reshape(d, 1), beta.reshape(d, 1), emb)


def kernel(concept_ids, table, W, b, gamma, beta):
    bsz, fields = concept_ids.shape
    d = table.shape[1]
    n = bsz * fields

    table128 = _repack(table.T, blk=4096)

    ids = concept_ids.T.reshape(n).astype(jnp.int32)
    ids3d = ids.reshape(n // 1024, 8, _IDXW)

    emb = _sc_gather(ids3d, table128)

    w128 = jnp.concatenate([W, jnp.zeros_like(W)], axis=1)
    out_t = _lin_ln_t(emb, w128, b, gamma, beta, fields, bsz, blk=4096)
    return out_t.transpose(2, 0, 1)


# speedup vs baseline: 3.0113x; 1.1943x over previous
import functools

import jax
import jax.numpy as jnp
from jax import lax
from jax.experimental import pallas as pl
from jax.experimental.pallas import tpu as pltpu
from jax.experimental.pallas import tpu_sc as plsc

_LN_EPS = 1e-5
_NC = 2
_NS = 16
_NW = _NC * _NS
_IDXW = 128


def _repack_body(tt_ref, eye2_ref, out_ref):
    x = tt_ref[...]
    out_ref[...] = lax.dot_general(x, eye2_ref[...], (((0,), (0,)), ((), ())),
                                   preferred_element_type=jnp.float32)


def _repack(table_t, blk):
    d, v = table_t.shape
    eye2 = jnp.concatenate([jnp.eye(d, dtype=jnp.float32)] * 2, axis=1)
    return pl.pallas_call(
        _repack_body,
        grid=(pl.cdiv(v, blk),),
        in_specs=[
            pl.BlockSpec((d, blk), lambda i: (0, i)),
            pl.BlockSpec((d, 2 * d), lambda i: (0, 0)),
        ],
        out_specs=pl.BlockSpec((blk, 2 * d), lambda i: (i, 0)),
        out_shape=jax.ShapeDtypeStruct((v, 2 * d), jnp.float32),
    )(table_t, eye2)


def _gather_body(nchunks, ids_hbm, table_hbm, out_hbm, idx_v, rows_v, sem):
    wid = lax.axis_index("s") * _NC + lax.axis_index("c")

    def step(i, carry):
        ci = wid * nchunks + i
        pltpu.sync_copy(ids_hbm.at[ci], idx_v)
        for half in range(2):
            copies = []
            for j in range(4):
                cp = pltpu.make_async_copy(
                    table_hbm.at[idx_v.at[half * 4 + j]],
                    rows_v.at[pl.ds(j * _IDXW, _IDXW)],
                    sem,
                )
                cp.start()
                copies.append(cp)
            for cp in copies:
                cp.wait()
            off = pl.multiple_of((ci * 2 + half) * 512, 512)
            pltpu.sync_copy(rows_v, out_hbm.at[pl.ds(off, 512)])
        return carry

    lax.fori_loop(0, nchunks, step, 0)


def _sc_gather(ids3d, table128):
    n = ids3d.shape[0] * 1024
    nchunks = ids3d.shape[0] // _NW
    mesh = plsc.VectorSubcoreMesh(core_axis_name="c", subcore_axis_name="s")
    f = pl.kernel(
        functools.partial(_gather_body, nchunks),
        out_type=jax.ShapeDtypeStruct((n, 128), jnp.float32),
        mesh=mesh,
        scratch_types=[
            pltpu.VMEM((8, _IDXW), jnp.int32),
            pltpu.VMEM((512, 128), jnp.float32),
            pltpu.SemaphoreType.DMA,
        ],
    )
    return f(ids3d, table128)


def _lin_ln_t_body(w_ref, b_ref, g_ref, be_ref, emb_ref, out_ref):
    x = emb_ref[...]
    w = w_ref[...]
    y = lax.dot_general(w, x, (((1,), (1,)), ((), ())),
                        preferred_element_type=jnp.float32)
    y = y + b_ref[...]
    m = jnp.mean(y, axis=0, keepdims=True)
    c = y - m
    v = jnp.mean(c * c, axis=0, keepdims=True)
    r = (c * lax.rsqrt(v + _LN_EPS)) * g_ref[...] + be_ref[...]
    out_ref[...] = r[None]


def _lin_ln_t(emb, w128, b, gamma, beta, fields, bsz, blk):
    d = w128.shape[0]
    nb = bsz // blk
    return pl.pallas_call(
        _lin_ln_t_body,
        grid=(fields, nb),
        in_specs=[
            pl.BlockSpec((d, 2 * d), lambda f, i: (0, 0)),
            pl.BlockSpec((d, 1), lambda f, i: (0, 0)),
            pl.BlockSpec((d, 1), lambda f, i: (0, 0)),
            pl.BlockSpec((d, 1), lambda f, i: (0, 0)),
            pl.BlockSpec((blk, 2 * d), lambda f, i: (f * nb + i, 0)),
        ],
        out_specs=pl.BlockSpec((1, d, blk), lambda f, i: (f, 0, i)),
        out_shape=jax.ShapeDtypeStruct((fields, d, bsz), jnp.float32),
    )(w128, b.reshape(d, 1), gamma.reshape(d, 1), beta.reshape(d, 1), emb)


def kernel(concept_ids, table, W, b, gamma, beta):
    bsz, fields = concept_ids.shape
    d = table.shape[1]
    n = bsz * fields

    table128 = _repack(table.T, blk=8192)

    ids = concept_ids.T.reshape(n).astype(jnp.int32)
    ids3d = ids.reshape(n // 1024, 8, _IDXW)

    emb = _sc_gather(ids3d, table128)

    w128 = jnp.concatenate([W, jnp.zeros_like(W)], axis=1)
    out_t = _lin_ln_t(emb, w128, b, gamma, beta, fields, bsz, blk=8192)
    return out_t.transpose(2, 0, 1)


# speedup vs baseline: 3.1836x; 1.0572x over previous
import functools

import jax
import jax.numpy as jnp
from jax import lax
from jax.experimental import pallas as pl
from jax.experimental.pallas import tpu as pltpu
from jax.experimental.pallas import tpu_sc as plsc

_LN_EPS = 1e-5
_NC = 2
_NS = 16
_NW = _NC * _NS
_IDXW = 128


def _repack_body(tt_ref, eye2_ref, out_ref):
    x = tt_ref[...]
    out_ref[...] = lax.dot_general(x, eye2_ref[...], (((0,), (0,)), ((), ())),
                                   preferred_element_type=jnp.float32)


def _repack(table_t, blk):
    d, v = table_t.shape
    eye2 = jnp.concatenate([jnp.eye(d, dtype=jnp.float32)] * 2, axis=1)
    return pl.pallas_call(
        _repack_body,
        grid=(pl.cdiv(v, blk),),
        in_specs=[
            pl.BlockSpec((d, blk), lambda i: (0, i)),
            pl.BlockSpec((d, 2 * d), lambda i: (0, 0)),
        ],
        out_specs=pl.BlockSpec((blk, 2 * d), lambda i: (i, 0)),
        out_shape=jax.ShapeDtypeStruct((v, 2 * d), jnp.float32),
    )(table_t, eye2)


def _gather_body(nchunks, ids_hbm, table_hbm, out_hbm, idx_v, rows_v, sem):
    wid = lax.axis_index("s") * _NC + lax.axis_index("c")

    def step(i, carry):
        ci = wid * nchunks + i
        pltpu.sync_copy(ids_hbm.at[ci], idx_v)
        for half in range(2):
            copies = []
            for j in range(4):
                cp = pltpu.make_async_copy(
                    table_hbm.at[idx_v.at[half * 4 + j]],
                    rows_v.at[pl.ds(j * _IDXW, _IDXW)],
                    sem,
                )
                cp.start()
                copies.append(cp)
            for cp in copies:
                cp.wait()
            off = pl.multiple_of((ci * 2 + half) * 512, 512)
            pltpu.sync_copy(rows_v, out_hbm.at[pl.ds(off, 512)])
        return carry

    lax.fori_loop(0, nchunks, step, 0)


def _sc_gather(ids3d, table128):
    n = ids3d.shape[0] * 1024
    nchunks = ids3d.shape[0] // _NW
    mesh = plsc.VectorSubcoreMesh(core_axis_name="c", subcore_axis_name="s")
    f = pl.kernel(
        functools.partial(_gather_body, nchunks),
        out_type=jax.ShapeDtypeStruct((n, 128), jnp.float32),
        mesh=mesh,
        scratch_types=[
            pltpu.VMEM((8, _IDXW), jnp.int32),
            pltpu.VMEM((512, 128), jnp.float32),
            pltpu.SemaphoreType.DMA,
        ],
    )
    return f(ids3d, table128)


def _lin_ln_t_body(w_ref, b_ref, g_ref, be_ref, emb_ref, out_ref):
    x = emb_ref[...]
    w = w_ref[...]
    y = lax.dot_general(w, x, (((1,), (1,)), ((), ())),
                        preferred_element_type=jnp.float32)
    y = y + b_ref[...]
    m = jnp.mean(y, axis=0, keepdims=True)
    c = y - m
    v = jnp.mean(c * c, axis=0, keepdims=True)
    r = (c * lax.rsqrt(v + _LN_EPS)) * g_ref[...] + be_ref[...]
    out_ref[...] = r[None]


def _lin_ln_t(emb, w128, b, gamma, beta, fields, bsz, blk):
    d = w128.shape[0]
    nb = bsz // blk
    return pl.pallas_call(
        _lin_ln_t_body,
        grid=(fields, nb),
        in_specs=[
            pl.BlockSpec((d, 2 * d), lambda f, i: (0, 0)),
            pl.BlockSpec((d, 1), lambda f, i: (0, 0)),
            pl.BlockSpec((d, 1), lambda f, i: (0, 0)),
            pl.BlockSpec((d, 1), lambda f, i: (0, 0)),
            pl.BlockSpec((blk, 2 * d), lambda f, i: (f * nb + i, 0)),
        ],
        out_specs=pl.BlockSpec((1, d, blk), lambda f, i: (f, 0, i)),
        out_shape=jax.ShapeDtypeStruct((fields, d, bsz), jnp.float32),
    )(w128, b.reshape(d, 1), gamma.reshape(d, 1), beta.reshape(d, 1), emb)


def kernel(concept_ids, table, W, b, gamma, beta):
    bsz, fields = concept_ids.shape
    d = table.shape[1]
    n = bsz * fields

    table128 = _repack(table.T, blk=16384)

    ids = concept_ids.T.reshape(n).astype(jnp.int32)
    ids3d = ids.reshape(n // 1024, 8, _IDXW)

    emb = _sc_gather(ids3d, table128)

    w128 = jnp.concatenate([W, jnp.zeros_like(W)], axis=1)
    out_t = _lin_ln_t(emb, w128, b, gamma, beta, fields, bsz, blk=16384)
    return out_t.transpose(2, 0, 1)


# speedup vs baseline: 3.2393x; 1.0175x over previous
import functools

import jax
import jax.numpy as jnp
from jax import lax
from jax.experimental import pallas as pl
from jax.experimental.pallas import tpu as pltpu
from jax.experimental.pallas import tpu_sc as plsc

_LN_EPS = 1e-5
_NC = 2
_NS = 16
_NW = _NC * _NS
_IDXW = 128


def _repack_body(tt_ref, eye2_ref, out_ref):
    x = tt_ref[...]
    out_ref[...] = lax.dot_general(x, eye2_ref[...], (((0,), (0,)), ((), ())),
                                   preferred_element_type=jnp.float32)


def _repack(table_t, blk):
    d, v = table_t.shape
    eye2 = jnp.concatenate([jnp.eye(d, dtype=jnp.float32)] * 2, axis=1)
    return pl.pallas_call(
        _repack_body,
        grid=(pl.cdiv(v, blk),),
        in_specs=[
            pl.BlockSpec((d, blk), lambda i: (0, i)),
            pl.BlockSpec((d, 2 * d), lambda i: (0, 0)),
        ],
        out_specs=pl.BlockSpec((blk, 2 * d), lambda i: (i, 0)),
        out_shape=jax.ShapeDtypeStruct((v, 2 * d), jnp.float32),
    )(table_t, eye2)


def _gather_body(nchunks, ids_hbm, table_hbm, out_hbm, idx_v, rows_a, rows_b,
                 sem_g, sem_wa, sem_wb):
    wid = lax.axis_index("s") * _NC + lax.axis_index("c")
    base = wid * nchunks * 1024

    def step(c, carry):
        pltpu.sync_copy(ids_hbm.at[wid * nchunks + c], idx_v)
        for ph in range(4):
            buf = rows_a if ph % 2 == 0 else rows_b
            semw = sem_wa if ph % 2 == 0 else sem_wb

            @pl.when((c > 0) | (ph >= 2))
            def _(buf=buf, semw=semw):
                pltpu.make_async_copy(out_hbm.at[pl.ds(0, 256)], buf, semw).wait()

            copies = []
            for j in range(2):
                cp = pltpu.make_async_copy(
                    table_hbm.at[idx_v.at[ph * 2 + j]],
                    buf.at[pl.ds(j * _IDXW, _IDXW)],
                    sem_g,
                )
                cp.start()
                copies.append(cp)
            for cp in copies:
                cp.wait()
            off = pl.multiple_of(base + c * 1024 + ph * 256, 256)
            pltpu.make_async_copy(buf, out_hbm.at[pl.ds(off, 256)], semw).start()
        return carry

    lax.fori_loop(0, nchunks, step, 0)
    pltpu.make_async_copy(out_hbm.at[pl.ds(0, 256)], rows_a, sem_wa).wait()
    pltpu.make_async_copy(out_hbm.at[pl.ds(0, 256)], rows_b, sem_wb).wait()


def _sc_gather(ids3d, table128):
    n = ids3d.shape[0] * 1024
    nchunks = ids3d.shape[0] // _NW
    mesh = plsc.VectorSubcoreMesh(core_axis_name="c", subcore_axis_name="s")
    f = pl.kernel(
        functools.partial(_gather_body, nchunks),
        out_type=jax.ShapeDtypeStruct((n, 128), jnp.float32),
        mesh=mesh,
        scratch_types=[
            pltpu.VMEM((8, _IDXW), jnp.int32),
            pltpu.VMEM((256, 128), jnp.float32),
            pltpu.VMEM((256, 128), jnp.float32),
            pltpu.SemaphoreType.DMA,
            pltpu.SemaphoreType.DMA,
            pltpu.SemaphoreType.DMA,
        ],
    )
    return f(ids3d, table128)


def _lin_ln_t_body(w_ref, b_ref, g_ref, be_ref, emb_ref, out_ref):
    x = emb_ref[...]
    w = w_ref[...]
    y = lax.dot_general(w, x, (((1,), (1,)), ((), ())),
                        preferred_element_type=jnp.float32)
    y = y + b_ref[...]
    m = jnp.mean(y, axis=0, keepdims=True)
    c = y - m
    v = jnp.mean(c * c, axis=0, keepdims=True)
    r = (c * lax.rsqrt(v + _LN_EPS)) * g_ref[...] + be_ref[...]
    out_ref[...] = r[None]


def _lin_ln_t(emb, w128, b, gamma, beta, fields, bsz, blk):
    d = w128.shape[0]
    nb = bsz // blk
    return pl.pallas_call(
        _lin_ln_t_body,
        grid=(fields, nb),
        in_specs=[
            pl.BlockSpec((d, 2 * d), lambda f, i: (0, 0)),
            pl.BlockSpec((d, 1), lambda f, i: (0, 0)),
            pl.BlockSpec((d, 1), lambda f, i: (0, 0)),
            pl.BlockSpec((d, 1), lambda f, i: (0, 0)),
            pl.BlockSpec((blk, 2 * d), lambda f, i: (f * nb + i, 0)),
        ],
        out_specs=pl.BlockSpec((1, d, blk), lambda f, i: (f, 0, i)),
        out_shape=jax.ShapeDtypeStruct((fields, d, bsz), jnp.float32),
    )(w128, b.reshape(d, 1), gamma.reshape(d, 1), beta.reshape(d, 1), emb)


def kernel(concept_ids, table, W, b, gamma, beta):
    bsz, fields = concept_ids.shape
    d = table.shape[1]
    n = bsz * fields

    table128 = _repack(table.T, blk=16384)

    ids = concept_ids.T.reshape(n).astype(jnp.int32)
    ids3d = ids.reshape(n // 1024, 8, _IDXW)

    emb = _sc_gather(ids3d, table128)

    w128 = jnp.concatenate([W, jnp.zeros_like(W)], axis=1)
    out_t = _lin_ln_t(emb, w128, b, gamma, beta, fields, bsz, blk=16384)
    return out_t.transpose(2, 0, 1)


# speedup vs baseline: 3.8055x; 1.1748x over previous
import functools

import jax
import jax.numpy as jnp
from jax import lax
from jax.experimental import pallas as pl
from jax.experimental.pallas import tpu as pltpu
from jax.experimental.pallas import tpu_sc as plsc

_LN_EPS = 1e-5
_NC = 2
_NS = 16
_NW = _NC * _NS
_IDXW = 128


def _repack_body(lo_ref, hi_ref, eye_ref, out_ref):
    x12 = jnp.concatenate([lo_ref[...], hi_ref[...]], axis=0)
    out_ref[...] = lax.dot_general(x12, eye_ref[...], (((0,), (0,)), ((), ())),
                                   preferred_element_type=jnp.float32)


def _repack(table_t, vh, blk):
    d, v = table_t.shape
    nbh = vh // blk
    vlast = -(-v // blk) - 1
    eye = jnp.eye(2 * d, dtype=jnp.float32)
    return pl.pallas_call(
        _repack_body,
        grid=(nbh,),
        in_specs=[
            pl.BlockSpec((d, blk), lambda i: (0, i)),
            pl.BlockSpec((d, blk), lambda i: (0, jnp.minimum(i + nbh, vlast))),
            pl.BlockSpec((2 * d, 2 * d), lambda i: (0, 0)),
        ],
        out_specs=pl.BlockSpec((blk, 2 * d), lambda i: (i, 0)),
        out_shape=jax.ShapeDtypeStruct((vh, 2 * d), jnp.float32),
    )(table_t, table_t, eye)


def _gather_body(nchunks, ids_hbm, table_hbm, out_hbm, idx_v, rows_a, rows_b,
                 sem_g, sem_wa, sem_wb):
    wid = lax.axis_index("s") * _NC + lax.axis_index("c")
    base = wid * nchunks * 1024

    def step(c, carry):
        pltpu.sync_copy(ids_hbm.at[wid * nchunks + c], idx_v)
        for ph in range(4):
            buf = rows_a if ph % 2 == 0 else rows_b
            semw = sem_wa if ph % 2 == 0 else sem_wb

            @pl.when((c > 0) | (ph >= 2))
            def _(buf=buf, semw=semw):
                pltpu.make_async_copy(out_hbm.at[pl.ds(0, 256)], buf, semw).wait()

            copies = []
            for j in range(2):
                cp = pltpu.make_async_copy(
                    table_hbm.at[idx_v.at[ph * 2 + j]],
                    buf.at[pl.ds(j * _IDXW, _IDXW)],
                    sem_g,
                )
                cp.start()
                copies.append(cp)
            for cp in copies:
                cp.wait()
            off = pl.multiple_of(base + c * 1024 + ph * 256, 256)
            pltpu.make_async_copy(buf, out_hbm.at[pl.ds(off, 256)], semw).start()
        return carry

    lax.fori_loop(0, nchunks, step, 0)
    pltpu.make_async_copy(out_hbm.at[pl.ds(0, 256)], rows_a, sem_wa).wait()
    pltpu.make_async_copy(out_hbm.at[pl.ds(0, 256)], rows_b, sem_wb).wait()


def _sc_gather(ids3d, table128):
    n = ids3d.shape[0] * 1024
    nchunks = ids3d.shape[0] // _NW
    mesh = plsc.VectorSubcoreMesh(core_axis_name="c", subcore_axis_name="s")
    f = pl.kernel(
        functools.partial(_gather_body, nchunks),
        out_type=jax.ShapeDtypeStruct((n, 128), jnp.float32),
        mesh=mesh,
        scratch_types=[
            pltpu.VMEM((8, _IDXW), jnp.int32),
            pltpu.VMEM((256, 128), jnp.float32),
            pltpu.VMEM((256, 128), jnp.float32),
            pltpu.SemaphoreType.DMA,
            pltpu.SemaphoreType.DMA,
            pltpu.SemaphoreType.DMA,
        ],
    )
    return f(ids3d, table128)


def _lin_ln_t_body(w_ref, b_ref, g_ref, be_ref, p_ref, emb_ref, out_ref):
    x = emb_ref[...]
    w = w_ref[...]
    y01 = lax.dot_general(w, x, (((1,), (1,)), ((), ())),
                          preferred_element_type=jnp.float32)
    p = p_ref[0]
    y = jnp.where(p > 0.5, y01[64:128], y01[0:64])
    y = y + b_ref[...]
    m = jnp.mean(y, axis=0, keepdims=True)
    c = y - m
    v = jnp.mean(c * c, axis=0, keepdims=True)
    r = (c * lax.rsqrt(v + _LN_EPS)) * g_ref[...] + be_ref[...]
    out_ref[...] = r[None]


def _lin_ln_t(emb, w_cat, b, gamma, beta, p2d, fields, bsz, blk):
    d = w_cat.shape[0] // 2
    nb = bsz // blk
    return pl.pallas_call(
        _lin_ln_t_body,
        grid=(fields, nb),
        in_specs=[
            pl.BlockSpec((2 * d, 2 * d), lambda f, i: (0, 0)),
            pl.BlockSpec((d, 1), lambda f, i: (0, 0)),
            pl.BlockSpec((d, 1), lambda f, i: (0, 0)),
            pl.BlockSpec((d, 1), lambda f, i: (0, 0)),
            pl.BlockSpec((1, 1, blk), lambda f, i: (f, 0, i)),
            pl.BlockSpec((blk, 2 * d), lambda f, i: (f * nb + i, 0)),
        ],
        out_specs=pl.BlockSpec((1, d, blk), lambda f, i: (f, 0, i)),
        out_shape=jax.ShapeDtypeStruct((fields, d, bsz), jnp.float32),
    )(w_cat, b.reshape(d, 1), gamma.reshape(d, 1), beta.reshape(d, 1), p2d, emb)


def kernel(concept_ids, table, W, b, gamma, beta):
    bsz, fields = concept_ids.shape
    d = table.shape[1]
    n = bsz * fields
    v = table.shape[0]

    blk = 8192
    nbh = -(-v // (2 * blk))
    vh = nbh * blk
    table128 = _repack(table.T, vh, blk)

    ids_t = concept_ids.T.astype(jnp.int32)
    p2d = (ids_t >= vh).astype(jnp.float32).reshape(fields, 1, bsz)
    idq = jnp.where(ids_t >= vh, ids_t - vh, ids_t).reshape(n)
    ids3d = idq.reshape(n // 1024, 8, _IDXW)

    emb = _sc_gather(ids3d, table128)

    z = jnp.zeros_like(W)
    w_cat = jnp.concatenate(
        [jnp.concatenate([W, z], axis=1), jnp.concatenate([z, W], axis=1)],
        axis=0,
    )
    out_t = _lin_ln_t(emb, w_cat, b, gamma, beta, p2d, fields, bsz, blk=16384)
    return out_t.transpose(2, 0, 1)


# speedup vs baseline: 3.8387x; 1.0087x over previous
import functools

import jax
import jax.numpy as jnp
from jax import lax
from jax.experimental import pallas as pl
from jax.experimental.pallas import tpu as pltpu
from jax.experimental.pallas import tpu_sc as plsc

_LN_EPS = 1e-5
_NC = 2
_NS = 16
_NW = _NC * _NS
_IDXW = 128


def _repack_body(lo_ref, hi_ref, eye_ref, out_ref):
    x12 = jnp.concatenate([lo_ref[...], hi_ref[...]], axis=0)
    out_ref[...] = lax.dot_general(x12, eye_ref[...], (((0,), (0,)), ((), ())),
                                   preferred_element_type=jnp.float32)


def _repack(table_t, vh, blk):
    d, v = table_t.shape
    nbh = vh // blk
    vlast = -(-v // blk) - 1
    eye = jnp.eye(2 * d, dtype=jnp.float32)
    return pl.pallas_call(
        _repack_body,
        grid=(nbh,),
        in_specs=[
            pl.BlockSpec((d, blk), lambda i: (0, i)),
            pl.BlockSpec((d, blk), lambda i: (0, jnp.minimum(i + nbh, vlast))),
            pl.BlockSpec((2 * d, 2 * d), lambda i: (0, 0)),
        ],
        out_specs=pl.BlockSpec((blk, 2 * d), lambda i: (i, 0)),
        out_shape=jax.ShapeDtypeStruct((vh, 2 * d), jnp.float32),
    )(table_t, table_t, eye)


def _gather_body(nchunks, ids_hbm, table_hbm, out_hbm, idx_v, rows_a, rows_b,
                 sem_ga, sem_gb, sem_wa, sem_wb):
    wid = lax.axis_index("s") * _NC + lax.axis_index("c")
    base = wid * nchunks * 1024
    bufs = (rows_a, rows_b)
    sems_g = (sem_ga, sem_gb)
    sems_w = (sem_wa, sem_wb)

    def fire(buf, sem, rows):
        for j in range(2):
            pltpu.make_async_copy(
                table_hbm.at[idx_v.at[rows + j]],
                buf.at[pl.ds(j * _IDXW, _IDXW)],
                sem,
            ).start()

    def drain_g(buf, sem):
        for j in range(2):
            pltpu.make_async_copy(
                table_hbm.at[idx_v.at[j]], buf.at[pl.ds(j * _IDXW, _IDXW)], sem
            ).wait()

    def drain_w(buf, sem):
        pltpu.make_async_copy(out_hbm.at[pl.ds(0, 256)], buf, sem).wait()

    def step(c, carry):
        pltpu.sync_copy(ids_hbm.at[wid * nchunks + c], idx_v)
        for ph in range(4):
            pi = ph % 2
            qi = (ph - 1) % 2
            @pl.when((c > 0) | (ph >= 2))
            def _(pi=pi):
                drain_w(bufs[pi], sems_w[pi])

            fire(bufs[pi], sems_g[pi], ph * 2)
            @pl.when((c > 0) | (ph >= 1))
            def _(c=c, ph=ph, qi=qi):
                drain_g(bufs[qi], sems_g[qi])
                pidx = c * 4 + ph - 1
                off = pl.multiple_of(base + pidx * 256, 256)
                pltpu.make_async_copy(
                    bufs[qi], out_hbm.at[pl.ds(off, 256)], sems_w[qi]
                ).start()
        return carry

    lax.fori_loop(0, nchunks, step, 0)
    drain_g(rows_b, sem_gb)
    last = pl.multiple_of(base + (nchunks * 4 - 1) * 256, 256)
    pltpu.make_async_copy(rows_b, out_hbm.at[pl.ds(last, 256)], sem_wb).start()
    drain_w(rows_a, sem_wa)
    drain_w(rows_b, sem_wb)


def _sc_gather(ids3d, table128):
    n = ids3d.shape[0] * 1024
    nchunks = ids3d.shape[0] // _NW
    mesh = plsc.VectorSubcoreMesh(core_axis_name="c", subcore_axis_name="s")
    f = pl.kernel(
        functools.partial(_gather_body, nchunks),
        out_type=jax.ShapeDtypeStruct((n, 128), jnp.float32),
        mesh=mesh,
        scratch_types=[
            pltpu.VMEM((8, _IDXW), jnp.int32),
            pltpu.VMEM((256, 128), jnp.float32),
            pltpu.VMEM((256, 128), jnp.float32),
            pltpu.SemaphoreType.DMA,
            pltpu.SemaphoreType.DMA,
            pltpu.SemaphoreType.DMA,
            pltpu.SemaphoreType.DMA,
        ],
    )
    return f(ids3d, table128)


def _lin_ln_t_body(w_ref, b_ref, g_ref, be_ref, p_ref, emb_ref, out_ref):
    x = emb_ref[...]
    w = w_ref[...]
    y01 = lax.dot_general(w, x, (((1,), (1,)), ((), ())),
                          preferred_element_type=jnp.float32)
    p = p_ref[0]
    y = jnp.where(p > 0.5, y01[64:128], y01[0:64])
    y = y + b_ref[...]
    m = jnp.mean(y, axis=0, keepdims=True)
    c = y - m
    v = jnp.mean(c * c, axis=0, keepdims=True)
    r = (c * lax.rsqrt(v + _LN_EPS)) * g_ref[...] + be_ref[...]
    out_ref[...] = r[None]


def _lin_ln_t(emb, w_cat, b, gamma, beta, p2d, fields, bsz, blk):
    d = w_cat.shape[0] // 2
    nb = bsz // blk
    return pl.pallas_call(
        _lin_ln_t_body,
        grid=(fields, nb),
        in_specs=[
            pl.BlockSpec((2 * d, 2 * d), lambda f, i: (0, 0)),
            pl.BlockSpec((d, 1), lambda f, i: (0, 0)),
            pl.BlockSpec((d, 1), lambda f, i: (0, 0)),
            pl.BlockSpec((d, 1), lambda f, i: (0, 0)),
            pl.BlockSpec((1, 1, blk), lambda f, i: (f, 0, i)),
            pl.BlockSpec((blk, 2 * d), lambda f, i: (f * nb + i, 0)),
        ],
        out_specs=pl.BlockSpec((1, d, blk), lambda f, i: (f, 0, i)),
        out_shape=jax.ShapeDtypeStruct((fields, d, bsz), jnp.float32),
    )(w_cat, b.reshape(d, 1), gamma.reshape(d, 1), beta.reshape(d, 1), p2d, emb)


def kernel(concept_ids, table, W, b, gamma, beta):
    bsz, fields = concept_ids.shape
    d = table.shape[1]
    n = bsz * fields
    v = table.shape[0]

    blk = 8192
    nbh = -(-v // (2 * blk))
    vh = nbh * blk
    table128 = _repack(table.T, vh, blk)

    ids_t = concept_ids.T.astype(jnp.int32)
    p2d = (ids_t >= vh).astype(jnp.float32).reshape(fields, 1, bsz)
    idq = jnp.where(ids_t >= vh, ids_t - vh, ids_t).reshape(n)
    ids3d = idq.reshape(n // 1024, 8, _IDXW)

    emb = _sc_gather(ids3d, table128)

    z = jnp.zeros_like(W)
    w_cat = jnp.concatenate(
        [jnp.concatenate([W, z], axis=1), jnp.concatenate([z, W], axis=1)],
        axis=0,
    )
    out_t = _lin_ln_t(emb, w_cat, b, gamma, beta, p2d, fields, bsz, blk=16384)
    return out_t.transpose(2, 0, 1)
